# baseline (speedup 1.0000x reference)
import functools

import jax
import jax.numpy as jnp
from jax.experimental import pallas as pl

B = 4096
BLOCK_R = 256
_ROT_A = (13, 15, 26, 6)
_ROT_B = (17, 29, 16, 24)
_TINY = 1.1754943508222875e-38
_NEG_INF = float("-inf")


def _threefry_round(x0, x1, r):
    x0 = x0 + x1
    x1 = (x1 << r) | (x1 >> (32 - r))
    x1 = x0 ^ x1
    return x0, x1


def _threefry_bits(lo, key):
    ks0 = jnp.uint32(0)
    ks1 = jnp.uint32(key)
    ks2 = jnp.uint32(key ^ 0x1BD11BDA)
    x0 = jnp.zeros_like(lo)
    x1 = lo + ks1
    schedule = (
        (_ROT_A, ks1, ks2, 1),
        (_ROT_B, ks2, ks0, 2),
        (_ROT_A, ks0, ks1, 3),
        (_ROT_B, ks1, ks2, 4),
        (_ROT_A, ks2, ks0, 5),
    )
    for rots, a, b, c in schedule:
        for r in rots:
            x0, x1 = _threefry_round(x0, x1, r)
        x0 = x0 + a
        x1 = x1 + b + jnp.uint32(c)
    return x0 ^ x1


def _gumbel(bits):
    fb = (bits >> 9) | jnp.uint32(0x3F800000)
    f = jax.lax.bitcast_convert_type(fb, jnp.float32) - jnp.float32(1.0)
    u = jnp.maximum(f, jnp.float32(_TINY))
    return -jnp.log(-jnp.log(u))


def _sample_value(key_vals, d, colid):
    m = jnp.max(key_vals, axis=1, keepdims=True)
    idx = jnp.min(jnp.where(key_vals == m, colid, jnp.int32(B)), axis=1,
                  keepdims=True)
    return jnp.max(jnp.where(colid == idx, d, jnp.float32(_NEG_INF)), axis=1,
                   keepdims=True)


def _body(dist_ref, prow_ref, pcol_ref, out_ref):
    i = pl.program_id(0)
    d = dist_ref[...]
    mask = pcol_ref[...] == prow_ref[...]
    row = jax.lax.broadcasted_iota(jnp.uint32, (BLOCK_R, B), 0)
    col = jax.lax.broadcasted_iota(jnp.uint32, (BLOCK_R, B), 1)
    lo = ((jnp.uint32(i * BLOCK_R) + row) << 12) | col
    gp = _gumbel(_threefry_bits(lo, 123))
    gn = _gumbel(_threefry_bits(lo, 456))
    key_pos = jnp.where(mask, d + gp, jnp.float32(_NEG_INF))
    key_neg = jnp.where(mask, jnp.float32(_NEG_INF), gn - d)
    colid = jax.lax.broadcasted_iota(jnp.int32, (BLOCK_R, B), 1)
    pos = _sample_value(key_pos, d, colid)
    neg = _sample_value(key_neg, d, colid)
    x = pos - neg
    out_ref[...] = jnp.maximum(x, 0.0) + jnp.log1p(jnp.exp(-jnp.abs(x)))


@jax.jit
def kernel(dist, pids):
    prow = pids.reshape(1, B)
    pcol = pids.reshape(B, 1)
    out = pl.pallas_call(
        _body,
        grid=(B // BLOCK_R,),
        in_specs=[
            pl.BlockSpec((BLOCK_R, B), lambda i: (i, 0)),
            pl.BlockSpec((1, B), lambda i: (0, 0)),
            pl.BlockSpec((BLOCK_R, 1), lambda i: (i, 0)),
        ],
        out_specs=pl.BlockSpec((BLOCK_R, 1), lambda i: (i, 0)),
        out_shape=jax.ShapeDtypeStruct((B, 1), jnp.float32),
    )(dist, prow, pcol)
    return out.reshape(B)


# speedup vs baseline: 2.1240x; 2.1240x over previous
import functools

import jax
import jax.numpy as jnp
from jax.experimental import pallas as pl

B = 4096
BLOCK_R = 256
_ROT_A = (13, 15, 26, 6)
_ROT_B = (17, 29, 16, 24)
_TINY = 1.1754943508222875e-38
_NEG_INF = float("-inf")


def _threefry_round(x0, x1, r):
    x0 = x0 + x1
    x1 = (x1 << r) | (x1 >> (32 - r))
    x1 = x0 ^ x1
    return x0, x1


def _threefry_bits(lo, ks1, ks2):
    k21 = ks2 + jnp.uint32(1)
    k13 = ks1 + jnp.uint32(3)
    k24 = ks2 + jnp.uint32(4)
    x1 = lo + ks1
    x0 = x1
    x1 = x1 ^ ((x1 << 13) | (x1 >> 19))
    for r in _ROT_A[1:]:
        x0, x1 = _threefry_round(x0, x1, r)
    x0 = x0 + ks1
    x1 = x1 + k21
    for r in _ROT_B:
        x0, x1 = _threefry_round(x0, x1, r)
    x0 = x0 + ks2
    x1 = x1 + jnp.uint32(2)
    for r in _ROT_A:
        x0, x1 = _threefry_round(x0, x1, r)
    x1 = x1 + k13
    for r in _ROT_B:
        x0, x1 = _threefry_round(x0, x1, r)
    x0 = x0 + ks1
    x1 = x1 + k24
    for r in _ROT_A:
        x0, x1 = _threefry_round(x0, x1, r)
    x0 = x0 + ks2
    x1 = x1 + jnp.uint32(5)
    return x0 ^ x1


def _gumbel(bits):
    fb = (bits >> 9) | jnp.uint32(0x3F800000)
    f = jax.lax.bitcast_convert_type(fb, jnp.float32) - jnp.float32(1.0)
    u = jnp.maximum(f, jnp.float32(_TINY))
    return -jnp.log(-jnp.log(u))


def _sample_value(key_vals, d, colid):
    m = jnp.max(key_vals, axis=1, keepdims=True)
    idx = jnp.min(jnp.where(key_vals == m, colid, jnp.int32(B)), axis=1,
                  keepdims=True)
    return jnp.max(jnp.where(colid == idx, d, jnp.float32(_NEG_INF)), axis=1,
                   keepdims=True)


def _body(dist_ref, prow_ref, pcol_ref, out_ref):
    i = pl.program_id(0)
    d = dist_ref[...]
    mask = pcol_ref[...] == prow_ref[...]
    row = jax.lax.broadcasted_iota(jnp.uint32, (BLOCK_R, B), 0)
    col = jax.lax.broadcasted_iota(jnp.uint32, (BLOCK_R, B), 1)
    lo = ((jnp.uint32(i * BLOCK_R) + row) << 12) | col
    ks1 = jnp.where(mask, jnp.uint32(123), jnp.uint32(456))
    ks2 = ks1 ^ jnp.uint32(0x1BD11BDA)
    g = _gumbel(_threefry_bits(lo, ks1, ks2))
    key_pos = jnp.where(mask, d + g, jnp.float32(_NEG_INF))
    key_neg = jnp.where(mask, jnp.float32(_NEG_INF), g - d)
    colid = jax.lax.broadcasted_iota(jnp.int32, (BLOCK_R, B), 1)
    pos = _sample_value(key_pos, d, colid)
    neg = _sample_value(key_neg, d, colid)
    x = pos - neg
    out_ref[...] = jnp.maximum(x, 0.0) + jnp.log1p(jnp.exp(-jnp.abs(x)))


@jax.jit
def kernel(dist, pids):
    prow = pids.reshape(1, B)
    pcol = pids.reshape(B, 1)
    out = pl.pallas_call(
        _body,
        grid=(B // BLOCK_R,),
        in_specs=[
            pl.BlockSpec((BLOCK_R, B), lambda i: (i, 0)),
            pl.BlockSpec((1, B), lambda i: (0, 0)),
            pl.BlockSpec((BLOCK_R, 1), lambda i: (i, 0)),
        ],
        out_specs=pl.BlockSpec((BLOCK_R, 1), lambda i: (i, 0)),
        out_shape=jax.ShapeDtypeStruct((B, 1), jnp.float32),
    )(dist, prow, pcol)
    return out.reshape(B)
